# Initial kernel scaffold; baseline (speedup 1.0000x reference)
import jax
import jax.numpy as jnp
from jax.experimental import pallas as pl


def kernel(x, edge_index, W_in, b_in, Wrel1, brel1, Wroot1, Wrel2, brel2, Wroot2):
    raise NotImplementedError("write your pallas kernel here")


# speedup vs baseline: 5.0585x; 5.0585x over previous
import functools

import jax
import jax.numpy as jnp
from jax import lax
from jax.experimental import pallas as pl
from jax.experimental.pallas import tpu as pltpu
from jax.experimental.pallas import tpu_sc as plsc

NC = 2
NS = 16
CHUNK = 80


def _spmm_partials(h, src, dst, zeros, n_pad):
    n, d = h.shape
    e = src.shape[0]
    rows_per_tile = n_pad // NS
    edges_per_tile = e // (NC * NS)
    n_chunks = edges_per_tile // CHUNK

    mesh = plsc.VectorSubcoreMesh(core_axis_name="c", subcore_axis_name="s")

    @functools.partial(
        pl.kernel,
        out_type=jax.ShapeDtypeStruct((NC, n_pad, d), jnp.float32),
        mesh=mesh,
        scratch_types=[
            pltpu.VMEM((CHUNK,), jnp.int32),
            pltpu.VMEM((CHUNK,), jnp.int32),
            pltpu.VMEM((CHUNK, d), jnp.float32),
            pltpu.VMEM_SHARED((n_pad, d), jnp.float32),
            pltpu.SemaphoreType.DMA,
        ],
    )
    def k(h_hbm, src_hbm, dst_hbm, zeros_hbm, out_hbm, idx_s, idx_d, rows,
          agg, sem):
        c = lax.axis_index("c")
        s = lax.axis_index("s")
        pltpu.sync_copy(zeros_hbm, agg.at[pl.ds(s * rows_per_tile,
                                                rows_per_tile)])
        plsc.subcore_barrier()
        base = (c * NS + s) * edges_per_tile

        def body(i, carry):
            off = base + i * CHUNK
            pltpu.sync_copy(src_hbm.at[pl.ds(off, CHUNK)], idx_s)
            pltpu.sync_copy(dst_hbm.at[pl.ds(off, CHUNK)], idx_d)
            pltpu.async_copy(h_hbm.at[idx_s], rows, sem).wait()
            pltpu.sync_copy(rows, agg.at[idx_d], add=True)
            return carry

        lax.fori_loop(0, n_chunks, body, 0)
        plsc.subcore_barrier()
        pltpu.sync_copy(
            agg.at[pl.ds(s * rows_per_tile, rows_per_tile)],
            out_hbm.at[c].at[pl.ds(s * rows_per_tile, rows_per_tile)])

    return k(h, src, dst, zeros)


_ROWS = 1000


def _dense_in(x, w, b):
    n, d = x.shape

    def body(x_ref, w_ref, b_ref, o_ref):
        acc = lax.dot_general(x_ref[...], w_ref[...],
                              (((1,), (1,)), ((), ())),
                              preferred_element_type=jnp.float32)
        o_ref[...] = jnp.maximum(acc + b_ref[...], 0.0)

    return pl.pallas_call(
        body,
        grid=(n // _ROWS,),
        in_specs=[
            pl.BlockSpec((_ROWS, d), lambda i: (i, 0)),
            pl.BlockSpec((d, d), lambda i: (0, 0)),
            pl.BlockSpec((1, d), lambda i: (0, 0)),
        ],
        out_specs=pl.BlockSpec((_ROWS, d), lambda i: (i, 0)),
        out_shape=jax.ShapeDtypeStruct((n, d), jnp.float32),
    )(x, w, b.reshape(1, d))


def _dense_layer(p, h, wrel, brel, wroot):
    n, d = h.shape

    def body(p_ref, h_ref, wrel_ref, brel_ref, wroot_ref, o_ref):
        agg = p_ref[0] + p_ref[1]
        acc = lax.dot_general(agg, wrel_ref[...], (((1,), (1,)), ((), ())),
                              preferred_element_type=jnp.float32)
        acc += lax.dot_general(h_ref[...], wroot_ref[...],
                               (((1,), (1,)), ((), ())),
                               preferred_element_type=jnp.float32)
        o_ref[...] = jnp.maximum(acc + brel_ref[...], 0.0)

    return pl.pallas_call(
        body,
        grid=(n // _ROWS,),
        in_specs=[
            pl.BlockSpec((2, _ROWS, d), lambda i: (0, i, 0)),
            pl.BlockSpec((_ROWS, d), lambda i: (i, 0)),
            pl.BlockSpec((d, d), lambda i: (0, 0)),
            pl.BlockSpec((1, d), lambda i: (0, 0)),
            pl.BlockSpec((d, d), lambda i: (0, 0)),
        ],
        out_specs=pl.BlockSpec((_ROWS, d), lambda i: (i, 0)),
        out_shape=jax.ShapeDtypeStruct((n, d), jnp.float32),
    )(p, h, wrel, brel.reshape(1, d), wroot)


def kernel(x, edge_index, W_in, b_in, Wrel1, brel1, Wroot1, Wrel2, brel2,
           Wroot2):
    n, d = x.shape
    src = edge_index[0]
    dst = edge_index[1]
    n_pad = ((n + 8 * NS - 1) // (8 * NS)) * (8 * NS)
    zeros = jnp.zeros((n_pad // NS, d), jnp.float32)

    h = _dense_in(x, W_in, b_in)
    p1 = _spmm_partials(h, src, dst, zeros, n_pad)
    h1 = _dense_layer(p1, h, Wrel1, brel1, Wroot1)
    p2 = _spmm_partials(h1, src, dst, zeros, n_pad)
    out = _dense_layer(p2, h1, Wrel2, brel2, Wroot2)
    return out
